# Initial kernel scaffold; baseline (speedup 1.0000x reference)
import functools

import jax
import jax.numpy as jnp
from jax import lax
from jax.experimental import pallas as pl
from jax.experimental.pallas import tpu as pltpu
from jax.experimental.pallas import tpu_sc as plsc

_N = 10000
_E = 320000
_R = 8
_F = 128
_H = 64
_C = 16

_NC = 2
_NS = 16
_NW = _NC * _NS
_CH = 128
_NCHUNK = 79
_EPT = _CH * _NCHUNK
_EPAD = _EPT * _NW
_PAD = _EPAD - _E
_CNT_ROWS = 1280
_ACC = 10016
_SLC = _ACC // _NS


def _sc_mesh():
    return plsc.VectorSubcoreMesh(core_axis_name="c", subcore_axis_name="s")


def _p1_body(src3, dst3, et3, zeros_hbm, ii3,
             fidx3, cidx3, cnt_part,
             src_v, dst_v, et_v, cnt_v, ii_v):
    cid = lax.axis_index("c")
    sid = lax.axis_index("s")
    w = sid * _NC + cid

    pltpu.sync_copy(src3.at[w], src_v)
    pltpu.sync_copy(dst3.at[w], dst_v)
    pltpu.sync_copy(et3.at[w], et_v)
    pltpu.sync_copy(zeros_hbm.at[pl.ds(0, _CNT_ROWS)], cnt_v)
    pltpu.sync_copy(ii3, ii_v)

    def _vec(j, _):
        def _vec16(k, _):
            sl = pl.ds(k * 16, 16)
            e = et_v[j, sl]
            src_v[j, sl] = src_v[j, sl] * _R + e
            dst_v[j, sl] = dst_v[j, sl] * _R + e
            return 0
        return lax.fori_loop(0, _CH // 16, _vec16, 0)
    lax.fori_loop(0, _NCHUNK, _vec, 0)

    def _cnt_chunk(j, _):
        def _cnt1(i, _):
            c = dst_v[j, i]
            row = c >> 6
            col = c & 63
            cnt_v[row, col] = cnt_v[row, col] + 1.0
            return 0
        return lax.fori_loop(0, _CH, _cnt1, 0)
    lax.fori_loop(0, _NCHUNK, _cnt_chunk, 0)

    pltpu.sync_copy(src_v, fidx3.at[w])
    pltpu.sync_copy(dst_v, cidx3.at[w])


    def _scoped(cnt_sh):
        @pl.when(sid == 0)
        def _():
            pltpu.sync_copy(zeros_hbm.at[pl.ds(0, _CNT_ROWS)], cnt_sh)
        plsc.subcore_barrier()

        def _red(j, _):
            pltpu.sync_copy(cnt_v.at[pl.ds(j * _CH, _CH)],
                            cnt_sh.at[ii_v.at[j]], add=True)
            return 0
        lax.fori_loop(0, _CNT_ROWS // _CH, _red, 0)
        plsc.subcore_barrier()
        rows = _CNT_ROWS // _NS
        pltpu.sync_copy(cnt_sh.at[pl.ds(sid * rows, rows)],
                        cnt_part.at[cid, pl.ds(sid * rows, rows)])

    pl.run_scoped(_scoped,
                  plsc.MemoryRef((_CNT_ROWS, 64), jnp.float32,
                                 pltpu.VMEM_SHARED))


def _p1_call(src3, dst3, et3, zeros_hbm, ii3):
    return pl.kernel(
        _p1_body,
        out_type=(
            jax.ShapeDtypeStruct((_NW, _NCHUNK, _CH), jnp.int32),
            jax.ShapeDtypeStruct((_NW, _NCHUNK, _CH), jnp.int32),
            jax.ShapeDtypeStruct((_NC, _CNT_ROWS, 64), jnp.float32),
        ),
        mesh=_sc_mesh(),
        scratch_types=[
            pltpu.VMEM((_NCHUNK, _CH), jnp.int32),
            pltpu.VMEM((_NCHUNK, _CH), jnp.int32),
            pltpu.VMEM((_NCHUNK, _CH), jnp.int32),
            pltpu.VMEM((_CNT_ROWS, 64), jnp.float32),
            pltpu.VMEM((_CNT_ROWS // _CH, _CH), jnp.int32),
        ],
    )(src3, dst3, et3, zeros_hbm, ii3)


def _p2_body(cnt_part, cidx3, norm3, cnt_v, tmp_v, cidx_v, norm_v):
    cid = lax.axis_index("c")
    sid = lax.axis_index("s")
    w = sid * _NC + cid

    pltpu.sync_copy(cnt_part.at[0], cnt_v)
    pltpu.sync_copy(cidx3.at[w], cidx_v)
    piece = 160
    for p in range(_CNT_ROWS // piece):
        pltpu.sync_copy(cnt_part.at[1, pl.ds(p * piece, piece)], tmp_v)

        def _acc(i, _):
            def _acc16(k, _):
                sl = pl.ds(k * 16, 16)
                cnt_v[p * piece + i, sl] = (cnt_v[p * piece + i, sl]
                                            + tmp_v[i, sl])
                return 0
            return lax.fori_loop(0, 4, _acc16, 0)
        lax.fori_loop(0, piece, _acc, 0)

    def _nrm(j, _):
        def _nrm16(k, _):
            sl = pl.ds(k * 16, 16)
            ci = cidx_v[j, sl]
            c = plsc.load_gather(cnt_v, [ci >> 6, ci & 63])
            norm_v[j, sl] = 1.0 / jnp.maximum(c, 1.0)
            return 0
        return lax.fori_loop(0, _CH // 16, _nrm16, 0)
    lax.fori_loop(0, _NCHUNK, _nrm, 0)

    pltpu.sync_copy(norm_v, norm3.at[w])


def _p2_call(cnt_part, cidx3):
    return pl.kernel(
        _p2_body,
        out_type=jax.ShapeDtypeStruct((_NW, _NCHUNK, _CH), jnp.float32),
        mesh=_sc_mesh(),
        scratch_types=[
            pltpu.VMEM((_CNT_ROWS, 64), jnp.float32),
            pltpu.VMEM((160, 64), jnp.float32),
            pltpu.VMEM((_NCHUNK, _CH), jnp.int32),
            pltpu.VMEM((_NCHUNK, _CH), jnp.float32),
        ],
    )(cnt_part, cidx3)


def _layer_body(table, fidx3, dst3, norm3, zeros_hbm,
                part,
                fidx_v, dst_v, norm_v, rows_v):
    cid = lax.axis_index("c")
    sid = lax.axis_index("s")
    w = sid * _NC + cid

    pltpu.sync_copy(fidx3.at[w], fidx_v)
    pltpu.sync_copy(dst3.at[w], dst_v)
    pltpu.sync_copy(norm3.at[w], norm_v)

    def _scoped(acc_sh):
        @pl.when(sid == 0)
        def _():
            pltpu.sync_copy(zeros_hbm, acc_sh)
        plsc.subcore_barrier()

        def _chunk(j, _):
            pltpu.sync_copy(table.at[fidx_v.at[j]], rows_v)

            def _scale(e, _):
                nv = norm_v[j, e]
                for k in range(_H // 16):
                    sl = pl.ds(k * 16, 16)
                    rows_v[e, sl] = rows_v[e, sl] * nv
                return 0
            lax.fori_loop(0, _CH, _scale, 0)
            pltpu.sync_copy(rows_v, acc_sh.at[dst_v.at[j]], add=True)
            return 0
        lax.fori_loop(0, _NCHUNK, _chunk, 0)

        plsc.subcore_barrier()
        pltpu.sync_copy(acc_sh.at[pl.ds(sid * _SLC, _SLC)],
                        part.at[cid, pl.ds(sid * _SLC, _SLC)])

    pl.run_scoped(_scoped,
                  plsc.MemoryRef((_ACC, _H), jnp.float32, pltpu.VMEM_SHARED))


def _layer_call(table, fidx3, dst3, norm3, zeros_hbm):
    return pl.kernel(
        _layer_body,
        out_type=jax.ShapeDtypeStruct((_NC, _ACC, _H), jnp.float32),
        mesh=_sc_mesh(),
        scratch_types=[
            pltpu.VMEM((_NCHUNK, _CH), jnp.int32),
            pltpu.VMEM((_NCHUNK, _CH), jnp.int32),
            pltpu.VMEM((_NCHUNK, _CH), jnp.float32),
            pltpu.VMEM((_CH, _H), jnp.float32),
        ],
    )(table, fidx3, dst3, norm3, zeros_hbm)


_BN = 1000


def _pre_body(x_ref, w_ref, root_ref, b_ref, t_ref, rp_ref):
    xb = x_ref[...]
    t_ref[...] = jnp.dot(xb, w_ref[...], preferred_element_type=jnp.float32)
    rp_ref[...] = (jnp.dot(xb, root_ref[...],
                           preferred_element_type=jnp.float32) + b_ref[...])


def _pre_call(x, wflat, root, b):
    f = x.shape[1]
    return pl.pallas_call(
        _pre_body,
        grid=(_N // _BN,),
        in_specs=[
            pl.BlockSpec((_BN, f), lambda i: (i, 0)),
            pl.BlockSpec((f, _R * _H), lambda i: (0, 0)),
            pl.BlockSpec((f, _H), lambda i: (0, 0)),
            pl.BlockSpec((1, _H), lambda i: (0, 0)),
        ],
        out_specs=[
            pl.BlockSpec((_BN, _R * _H), lambda i: (i, 0)),
            pl.BlockSpec((_BN, _H), lambda i: (i, 0)),
        ],
        out_shape=[
            jax.ShapeDtypeStruct((_N, _R * _H), jnp.float32),
            jax.ShapeDtypeStruct((_N, _H), jnp.float32),
        ],
    )(x, wflat, root, b)


def _mid_body(p0_ref, p1_ref, rp_ref, w_ref, root_ref, b_ref,
              h_ref, t_ref, rpn_ref):
    h = jnp.maximum(p0_ref[...] + p1_ref[...] + rp_ref[...], 0.0)
    h_ref[...] = h
    t_ref[...] = jnp.dot(h, w_ref[...], preferred_element_type=jnp.float32)
    rpn_ref[...] = (jnp.dot(h, root_ref[...],
                            preferred_element_type=jnp.float32) + b_ref[...])


def _mid_call(p0, p1, rp, wflat, root, b):
    return pl.pallas_call(
        _mid_body,
        grid=(_N // _BN,),
        in_specs=[
            pl.BlockSpec((_BN, _H), lambda i: (i, 0)),
            pl.BlockSpec((_BN, _H), lambda i: (i, 0)),
            pl.BlockSpec((_BN, _H), lambda i: (i, 0)),
            pl.BlockSpec((_H, _R * _H), lambda i: (0, 0)),
            pl.BlockSpec((_H, _H), lambda i: (0, 0)),
            pl.BlockSpec((1, _H), lambda i: (0, 0)),
        ],
        out_specs=[
            pl.BlockSpec((_BN, _H), lambda i: (i, 0)),
            pl.BlockSpec((_BN, _R * _H), lambda i: (i, 0)),
            pl.BlockSpec((_BN, _H), lambda i: (i, 0)),
        ],
        out_shape=[
            jax.ShapeDtypeStruct((_N, _H), jnp.float32),
            jax.ShapeDtypeStruct((_N, _R * _H), jnp.float32),
            jax.ShapeDtypeStruct((_N, _H), jnp.float32),
        ],
    )(p0, p1, rp, wflat, root, b)


def _post_body(p0_ref, p1_ref, rp_ref, h1_ref, h2_ref,
               lw1_ref, lw2_ref, lw3_ref, lb_ref, o_ref):
    h3 = jnp.maximum(p0_ref[...] + p1_ref[...] + rp_ref[...], 0.0)
    o_ref[...] = (
        jnp.dot(h1_ref[...], lw1_ref[...], preferred_element_type=jnp.float32)
        + jnp.dot(h2_ref[...], lw2_ref[...], preferred_element_type=jnp.float32)
        + jnp.dot(h3, lw3_ref[...], preferred_element_type=jnp.float32)
        + lb_ref[...])


def _post_call(p0, p1, rp, h1, h2, lw1, lw2, lw3, lb):
    return pl.pallas_call(
        _post_body,
        grid=(_N // _BN,),
        in_specs=[
            pl.BlockSpec((_BN, _H), lambda i: (i, 0)),
            pl.BlockSpec((_BN, _H), lambda i: (i, 0)),
            pl.BlockSpec((_BN, _H), lambda i: (i, 0)),
            pl.BlockSpec((_BN, _H), lambda i: (i, 0)),
            pl.BlockSpec((_BN, _H), lambda i: (i, 0)),
            pl.BlockSpec((_H, _C), lambda i: (0, 0)),
            pl.BlockSpec((_H, _C), lambda i: (0, 0)),
            pl.BlockSpec((_H, _C), lambda i: (0, 0)),
            pl.BlockSpec((1, _C), lambda i: (0, 0)),
        ],
        out_specs=pl.BlockSpec((_BN, _C), lambda i: (i, 0)),
        out_shape=jax.ShapeDtypeStruct((_N, _C), jnp.float32),
    )(p0, p1, rp, h1, h2, lw1, lw2, lw3, lb)


def kernel(x, edge_index, edge_type, w1, root1, b1, w2, root2, b2,
           w3, root3, b3, lin_w, lin_b):
    src = edge_index[0]
    dst = edge_index[1]

    src_p = jnp.concatenate([src, jnp.zeros((_PAD,), jnp.int32)])
    dst_p = jnp.concatenate([dst, jnp.full((_PAD,), _N, jnp.int32)])
    et_p = jnp.concatenate([edge_type, jnp.zeros((_PAD,), jnp.int32)])
    src3 = src_p.reshape(_NW, _NCHUNK, _CH)
    dst3 = dst_p.reshape(_NW, _NCHUNK, _CH)
    et3 = et_p.reshape(_NW, _NCHUNK, _CH)

    zeros_hbm = jnp.zeros((_ACC, _H), jnp.float32)
    ii3 = jnp.arange(_CNT_ROWS, dtype=jnp.int32).reshape(_CNT_ROWS // _CH, _CH)

    fidx3, cidx3, cnt_part = _p1_call(src3, dst3, et3, zeros_hbm, ii3)
    norm3 = _p2_call(cnt_part, cidx3)

    w1f = w1.transpose(1, 0, 2).reshape(_F, _R * _H)
    w2f = w2.transpose(1, 0, 2).reshape(_H, _R * _H)
    w3f = w3.transpose(1, 0, 2).reshape(_H, _R * _H)

    t1, rp1 = _pre_call(x, w1f, root1, b1.reshape(1, _H))
    pt1 = _layer_call(t1.reshape(_N * _R, _H), fidx3, dst3, norm3, zeros_hbm)
    h1, t2, rp2 = _mid_call(pt1[0, :_N], pt1[1, :_N], rp1, w2f, root2,
                            b2.reshape(1, _H))
    pt2 = _layer_call(t2.reshape(_N * _R, _H), fidx3, dst3, norm3, zeros_hbm)
    h2, t3, rp3 = _mid_call(pt2[0, :_N], pt2[1, :_N], rp2, w3f, root3,
                            b3.reshape(1, _H))
    pt3 = _layer_call(t3.reshape(_N * _R, _H), fidx3, dst3, norm3, zeros_hbm)
    out = _post_call(pt3[0, :_N], pt3[1, :_N], rp3, h1, h2,
                     lin_w[:_H], lin_w[_H:2 * _H], lin_w[2 * _H:],
                     lin_b.reshape(1, _C))
    return out


# speedup vs baseline: 19.0026x; 19.0026x over previous
import functools

import jax
import jax.numpy as jnp
from jax import lax
from jax.experimental import pallas as pl
from jax.experimental.pallas import tpu as pltpu
from jax.experimental.pallas import tpu_sc as plsc

_N = 10000
_E = 320000
_R = 8
_F = 128
_H = 64
_C = 16

_NC = 2
_NS = 16
_NW = _NC * _NS
_CH = 128
_NCHUNK = 79
_EPT = _CH * _NCHUNK
_EPAD = _EPT * _NW
_PAD = _EPAD - _E
_CNT = 80128
_CSL = _CNT // _NS
_ACC = 10112
_SLC = _ACC // _NS


def _sc_mesh():
    return plsc.VectorSubcoreMesh(core_axis_name="c", subcore_axis_name="s")


_SC_PARAMS = pltpu.CompilerParams(use_tc_tiling_on_sc=False)


def _p1_body(src3, dst3, et3, zeros16, ones16,
             fidx3, cidx3, cnt_out,
             src_v, dst_v, et_v, ones_v, cnt_sh):
    cid = lax.axis_index("c")
    sid = lax.axis_index("s")
    w = sid * _NC + cid

    pltpu.sync_copy(src3.at[w], src_v)
    pltpu.sync_copy(dst3.at[w], dst_v)
    pltpu.sync_copy(et3.at[w], et_v)
    pltpu.sync_copy(ones16, ones_v)
    pltpu.sync_copy(zeros16.at[pl.ds(sid * _CSL, _CSL)],
                    cnt_sh.at[pl.ds(sid * _CSL, _CSL)])

    def _vec(j, _):
        def _vec16(k, _):
            sl = pl.ds(k * 16, 16)
            e = et_v[j, sl]
            src_v[j, sl] = src_v[j, sl] * _R + e
            dst_v[j, sl] = dst_v[j, sl] * _R + e
            return 0
        return lax.fori_loop(0, _CH // 16, _vec16, 0)
    lax.fori_loop(0, _NCHUNK, _vec, 0)

    pltpu.sync_copy(src_v, fidx3.at[w])
    pltpu.sync_copy(dst_v, cidx3.at[w])

    plsc.subcore_barrier()

    def _cnt(j, _):
        pltpu.sync_copy(ones_v, cnt_sh.at[dst_v.at[j]], add=True)
        return 0
    lax.fori_loop(0, _NCHUNK, _cnt, 0)

    plsc.subcore_barrier()
    pltpu.sync_copy(cnt_sh.at[pl.ds(sid * _CSL, _CSL)],
                    cnt_out.at[cid, pl.ds(sid * _CSL, _CSL)])


def _p1_call(src3, dst3, et3, zeros16, ones16):
    return pl.kernel(
        _p1_body,
        out_type=(
            jax.ShapeDtypeStruct((_NW, _NCHUNK, _CH), jnp.int32),
            jax.ShapeDtypeStruct((_NW, _NCHUNK, _CH), jnp.int32),
            jax.ShapeDtypeStruct((_NC, _CNT, 16), jnp.float32),
        ),
        mesh=_sc_mesh(),
        compiler_params=_SC_PARAMS,
        scratch_types=[
            pltpu.VMEM((_NCHUNK, _CH), jnp.int32),
            pltpu.VMEM((_NCHUNK, _CH), jnp.int32),
            pltpu.VMEM((_NCHUNK, _CH), jnp.int32),
            pltpu.VMEM((_CH, 16), jnp.float32),
            pltpu.VMEM_SHARED((_CNT, 16), jnp.float32),
        ],
    )(src3, dst3, et3, zeros16, ones16)


def _norm_body(c_ref, o_ref):
    o_ref[...] = 1.0 / jnp.maximum(c_ref[0] + c_ref[1], 1.0)


def _norm_call(cnt_part):
    c = cnt_part.reshape(_NC, 313, 4096)
    out = pl.pallas_call(
        _norm_body,
        grid=(8,),
        in_specs=[pl.BlockSpec((_NC, 313, 512), lambda i: (0, 0, i))],
        out_specs=pl.BlockSpec((313, 512), lambda i: (0, i)),
        out_shape=jax.ShapeDtypeStruct((313, 4096), jnp.float32),
    )(c)
    return out.reshape(_CNT, 16)


def _layer_body(table, fidx3, dst3, cidx3, norm_tab, zeros_hbm,
                part,
                fidx_v, dst_v, cidx_v, nrm_v, rows_v, acc_sh):
    cid = lax.axis_index("c")
    sid = lax.axis_index("s")
    w = sid * _NC + cid

    pltpu.sync_copy(fidx3.at[w], fidx_v)
    pltpu.sync_copy(dst3.at[w], dst_v)
    pltpu.sync_copy(cidx3.at[w], cidx_v)

    @pl.when(sid == 0)
    def _():
        pltpu.sync_copy(zeros_hbm, acc_sh)
    plsc.subcore_barrier()

    def _chunk(j, _):
        pltpu.sync_copy(table.at[fidx_v.at[j]], rows_v)
        pltpu.sync_copy(norm_tab.at[cidx_v.at[j]], nrm_v)

        def _scale(e, _):
            nv = nrm_v[e, pl.ds(0, 16)]
            for k in range(_H // 16):
                sl = pl.ds(k * 16, 16)
                rows_v[e, sl] = rows_v[e, sl] * nv
            return 0
        lax.fori_loop(0, _CH, _scale, 0)
        pltpu.sync_copy(rows_v, acc_sh.at[dst_v.at[j]], add=True)
        return 0
    lax.fori_loop(0, _NCHUNK, _chunk, 0)

    plsc.subcore_barrier()
    pltpu.sync_copy(acc_sh.at[pl.ds(sid * _SLC, _SLC)],
                    part.at[cid, pl.ds(sid * _SLC, _SLC)])


def _layer_call(table, fidx3, dst3, cidx3, norm_tab, zeros_hbm):
    return pl.kernel(
        _layer_body,
        out_type=jax.ShapeDtypeStruct((_NC, _ACC, _H), jnp.float32),
        mesh=_sc_mesh(),
        compiler_params=_SC_PARAMS,
        scratch_types=[
            pltpu.VMEM((_NCHUNK, _CH), jnp.int32),
            pltpu.VMEM((_NCHUNK, _CH), jnp.int32),
            pltpu.VMEM((_NCHUNK, _CH), jnp.int32),
            pltpu.VMEM((_CH, 16), jnp.float32),
            pltpu.VMEM((_CH, _H), jnp.float32),
            pltpu.VMEM_SHARED((_ACC, _H), jnp.float32),
        ],
    )(table, fidx3, dst3, cidx3, norm_tab, zeros_hbm)


_BN = 1000


def _pre_body(x_ref, w_ref, root_ref, b_ref, t_ref, rp_ref):
    xb = x_ref[...]
    t_ref[...] = jnp.dot(xb, w_ref[...], preferred_element_type=jnp.float32)
    rp_ref[...] = (jnp.dot(xb, root_ref[...],
                           preferred_element_type=jnp.float32) + b_ref[...])


def _pre_call(x, wflat, root, b):
    f = x.shape[1]
    return pl.pallas_call(
        _pre_body,
        grid=(_N // _BN,),
        in_specs=[
            pl.BlockSpec((_BN, f), lambda i: (i, 0)),
            pl.BlockSpec((f, _R * _H), lambda i: (0, 0)),
            pl.BlockSpec((f, _H), lambda i: (0, 0)),
            pl.BlockSpec((1, _H), lambda i: (0, 0)),
        ],
        out_specs=[
            pl.BlockSpec((_BN, _R * _H), lambda i: (i, 0)),
            pl.BlockSpec((_BN, _H), lambda i: (i, 0)),
        ],
        out_shape=[
            jax.ShapeDtypeStruct((_N, _R * _H), jnp.float32),
            jax.ShapeDtypeStruct((_N, _H), jnp.float32),
        ],
    )(x, wflat, root, b)


def _mid_body(p0_ref, p1_ref, rp_ref, w_ref, root_ref, b_ref,
              h_ref, t_ref, rpn_ref):
    h = jnp.maximum(p0_ref[...] + p1_ref[...] + rp_ref[...], 0.0)
    h_ref[...] = h
    t_ref[...] = jnp.dot(h, w_ref[...], preferred_element_type=jnp.float32)
    rpn_ref[...] = (jnp.dot(h, root_ref[...],
                            preferred_element_type=jnp.float32) + b_ref[...])


def _mid_call(p0, p1, rp, wflat, root, b):
    return pl.pallas_call(
        _mid_body,
        grid=(_N // _BN,),
        in_specs=[
            pl.BlockSpec((_BN, _H), lambda i: (i, 0)),
            pl.BlockSpec((_BN, _H), lambda i: (i, 0)),
            pl.BlockSpec((_BN, _H), lambda i: (i, 0)),
            pl.BlockSpec((_H, _R * _H), lambda i: (0, 0)),
            pl.BlockSpec((_H, _H), lambda i: (0, 0)),
            pl.BlockSpec((1, _H), lambda i: (0, 0)),
        ],
        out_specs=[
            pl.BlockSpec((_BN, _H), lambda i: (i, 0)),
            pl.BlockSpec((_BN, _R * _H), lambda i: (i, 0)),
            pl.BlockSpec((_BN, _H), lambda i: (i, 0)),
        ],
        out_shape=[
            jax.ShapeDtypeStruct((_N, _H), jnp.float32),
            jax.ShapeDtypeStruct((_N, _R * _H), jnp.float32),
            jax.ShapeDtypeStruct((_N, _H), jnp.float32),
        ],
    )(p0, p1, rp, wflat, root, b)


def _post_body(p0_ref, p1_ref, rp_ref, h1_ref, h2_ref,
               lw1_ref, lw2_ref, lw3_ref, lb_ref, o_ref):
    h3 = jnp.maximum(p0_ref[...] + p1_ref[...] + rp_ref[...], 0.0)
    o_ref[...] = (
        jnp.dot(h1_ref[...], lw1_ref[...], preferred_element_type=jnp.float32)
        + jnp.dot(h2_ref[...], lw2_ref[...], preferred_element_type=jnp.float32)
        + jnp.dot(h3, lw3_ref[...], preferred_element_type=jnp.float32)
        + lb_ref[...])


def _post_call(p0, p1, rp, h1, h2, lw1, lw2, lw3, lb):
    return pl.pallas_call(
        _post_body,
        grid=(_N // _BN,),
        in_specs=[
            pl.BlockSpec((_BN, _H), lambda i: (i, 0)),
            pl.BlockSpec((_BN, _H), lambda i: (i, 0)),
            pl.BlockSpec((_BN, _H), lambda i: (i, 0)),
            pl.BlockSpec((_BN, _H), lambda i: (i, 0)),
            pl.BlockSpec((_BN, _H), lambda i: (i, 0)),
            pl.BlockSpec((_H, _C), lambda i: (0, 0)),
            pl.BlockSpec((_H, _C), lambda i: (0, 0)),
            pl.BlockSpec((_H, _C), lambda i: (0, 0)),
            pl.BlockSpec((1, _C), lambda i: (0, 0)),
        ],
        out_specs=pl.BlockSpec((_BN, _C), lambda i: (i, 0)),
        out_shape=jax.ShapeDtypeStruct((_N, _C), jnp.float32),
    )(p0, p1, rp, h1, h2, lw1, lw2, lw3, lb)


def kernel(x, edge_index, edge_type, w1, root1, b1, w2, root2, b2,
           w3, root3, b3, lin_w, lin_b):
    src = edge_index[0]
    dst = edge_index[1]

    src_p = jnp.concatenate([src, jnp.zeros((_PAD,), jnp.int32)])
    dst_p = jnp.concatenate([dst, jnp.full((_PAD,), _N, jnp.int32)])
    et_p = jnp.concatenate([edge_type, jnp.zeros((_PAD,), jnp.int32)])
    src3 = src_p.reshape(_NW, _NCHUNK, _CH)
    dst3 = dst_p.reshape(_NW, _NCHUNK, _CH)
    et3 = et_p.reshape(_NW, _NCHUNK, _CH)

    zeros_hbm = jnp.zeros((_ACC, _H), jnp.float32)
    zeros16 = jnp.zeros((_CNT, 16), jnp.float32)
    ones16 = jnp.ones((_CH, 16), jnp.float32)

    fidx3, cidx3, cnt_part = _p1_call(src3, dst3, et3, zeros16, ones16)
    norm_tab = _norm_call(cnt_part)

    w1f = w1.transpose(1, 0, 2).reshape(_F, _R * _H)
    w2f = w2.transpose(1, 0, 2).reshape(_H, _R * _H)
    w3f = w3.transpose(1, 0, 2).reshape(_H, _R * _H)

    t1, rp1 = _pre_call(x, w1f, root1, b1.reshape(1, _H))
    pt1 = _layer_call(t1.reshape(_N * _R, _H), fidx3, dst3, cidx3, norm_tab,
                      zeros_hbm)
    h1, t2, rp2 = _mid_call(pt1[0, :_N], pt1[1, :_N], rp1, w2f, root2,
                            b2.reshape(1, _H))
    pt2 = _layer_call(t2.reshape(_N * _R, _H), fidx3, dst3, cidx3, norm_tab,
                      zeros_hbm)
    h2, t3, rp3 = _mid_call(pt2[0, :_N], pt2[1, :_N], rp2, w3f, root3,
                            b3.reshape(1, _H))
    pt3 = _layer_call(t3.reshape(_N * _R, _H), fidx3, dst3, cidx3, norm_tab,
                      zeros_hbm)
    out = _post_call(pt3[0, :_N], pt3[1, :_N], rp3, h1, h2,
                     lin_w[:_H], lin_w[_H:2 * _H], lin_w[2 * _H:],
                     lin_b.reshape(1, _C))
    return out


# speedup vs baseline: 23.6903x; 1.2467x over previous
import functools

import jax
import jax.numpy as jnp
from jax import lax
from jax.experimental import pallas as pl
from jax.experimental.pallas import tpu as pltpu
from jax.experimental.pallas import tpu_sc as plsc

_N = 10000
_E = 320000
_R = 8
_F = 128
_H = 64
_C = 16

_NC = 2
_NS = 16
_NW = _NC * _NS
_CH = 128
_NCHUNK = 79
_EPT = _CH * _NCHUNK
_EPAD = _EPT * _NW
_PAD = _EPAD - _E
_CNT = 80128
_CSL = _CNT // _NS
_ACC = 10112
_SLC = _ACC // _NS


def _sc_mesh():
    return plsc.VectorSubcoreMesh(core_axis_name="c", subcore_axis_name="s")


_SC_PARAMS = pltpu.CompilerParams(use_tc_tiling_on_sc=False)


def _p1_body(src3, dst3, et3, zeros16, ones16,
             fidx3, cidx3, cnt_out,
             src_v, dst_v, et_v, ones_v, cnt_sh):
    cid = lax.axis_index("c")
    sid = lax.axis_index("s")
    w = sid * _NC + cid

    pltpu.sync_copy(src3.at[w], src_v)
    pltpu.sync_copy(dst3.at[w], dst_v)
    pltpu.sync_copy(et3.at[w], et_v)
    pltpu.sync_copy(ones16, ones_v)
    pltpu.sync_copy(zeros16.at[pl.ds(sid * _CSL, _CSL)],
                    cnt_sh.at[pl.ds(sid * _CSL, _CSL)])

    def _vec(j, _):
        def _vec16(k, _):
            sl = pl.ds(k * 16, 16)
            e = et_v[j, sl]
            src_v[j, sl] = src_v[j, sl] * _R + e
            dst_v[j, sl] = dst_v[j, sl] * _R + e
            return 0
        return lax.fori_loop(0, _CH // 16, _vec16, 0)
    lax.fori_loop(0, _NCHUNK, _vec, 0)

    pltpu.sync_copy(src_v, fidx3.at[w])
    pltpu.sync_copy(dst_v, cidx3.at[w])

    plsc.subcore_barrier()

    def _cnt(j, _):
        pltpu.sync_copy(ones_v, cnt_sh.at[dst_v.at[j]], add=True)
        return 0
    lax.fori_loop(0, _NCHUNK, _cnt, 0)

    plsc.subcore_barrier()
    pltpu.sync_copy(cnt_sh.at[pl.ds(sid * _CSL, _CSL)],
                    cnt_out.at[cid, pl.ds(sid * _CSL, _CSL)])


def _p1_call(src3, dst3, et3, zeros16, ones16):
    return pl.kernel(
        _p1_body,
        out_type=(
            jax.ShapeDtypeStruct((_NW, _NCHUNK, _CH), jnp.int32),
            jax.ShapeDtypeStruct((_NW, _NCHUNK, _CH), jnp.int32),
            jax.ShapeDtypeStruct((_NC, _CNT, 16), jnp.float32),
        ),
        mesh=_sc_mesh(),
        compiler_params=_SC_PARAMS,
        scratch_types=[
            pltpu.VMEM((_NCHUNK, _CH), jnp.int32),
            pltpu.VMEM((_NCHUNK, _CH), jnp.int32),
            pltpu.VMEM((_NCHUNK, _CH), jnp.int32),
            pltpu.VMEM((_CH, 16), jnp.float32),
            pltpu.VMEM_SHARED((_CNT, 16), jnp.float32),
        ],
    )(src3, dst3, et3, zeros16, ones16)


def _norm_body(c_ref, o_ref):
    o_ref[...] = 1.0 / jnp.maximum(c_ref[0] + c_ref[1], 1.0)


def _norm_call(cnt_part):
    c = cnt_part.reshape(_NC, 313, 4096)
    out = pl.pallas_call(
        _norm_body,
        grid=(8,),
        in_specs=[pl.BlockSpec((_NC, 313, 512), lambda i: (0, 0, i))],
        out_specs=pl.BlockSpec((313, 512), lambda i: (0, i)),
        out_shape=jax.ShapeDtypeStruct((313, 4096), jnp.float32),
    )(c)
    return out.reshape(_CNT, 16)


def _layer_body(table, fidx3, dst3, cidx3, norm_tab, zeros_hbm,
                part,
                fidx_v, dst_v, cidx_v, nrm_v, rows_v,
                sem_gr, sem_gn, sem_s, acc_sh):
    cid = lax.axis_index("c")
    sid = lax.axis_index("s")
    w = sid * _NC + cid

    pltpu.sync_copy(fidx3.at[w], fidx_v)
    pltpu.sync_copy(dst3.at[w], dst_v)
    pltpu.sync_copy(cidx3.at[w], cidx_v)

    pltpu.sync_copy(zeros_hbm.at[pl.ds(sid * _SLC, _SLC)],
                    acc_sh.at[pl.ds(sid * _SLC, _SLC)])
    plsc.subcore_barrier()

    def _fetch(j, b):
        pltpu.async_copy(table.at[fidx_v.at[j]], rows_v.at[b], sem_gr.at[b])
        pltpu.async_copy(norm_tab.at[cidx_v.at[j]], nrm_v.at[b], sem_gn.at[b])

    def _wait_fetch(j, b):
        pltpu.make_async_copy(table.at[fidx_v.at[j]], rows_v.at[b],
                              sem_gr.at[b]).wait()
        pltpu.make_async_copy(norm_tab.at[cidx_v.at[j]], nrm_v.at[b],
                              sem_gn.at[b]).wait()

    def _wait_scat(j, b):
        pltpu.make_async_copy(rows_v.at[b], acc_sh.at[dst_v.at[j]],
                              sem_s.at[b]).wait()

    for j in range(3):
        _fetch(j, j)

    def _chunk(j, _):
        b = lax.rem(j, 3)
        _wait_fetch(j, b)

        def _scale(e, _):
            nv = nrm_v[b, e, pl.ds(0, 16)]
            for k in range(_H // 16):
                sl = pl.ds(k * 16, 16)
                rows_v[b, e, sl] = rows_v[b, e, sl] * nv
            return 0
        lax.fori_loop(0, _CH, _scale, 0, unroll=8)

        b2 = lax.rem(j + 2, 3)

        @pl.when(j >= 1)
        def _():
            _wait_scat(j - 1, b2)

        @pl.when(j + 2 < _NCHUNK)
        def _():
            _fetch(j + 2, b2)

        pltpu.async_copy(rows_v.at[b], acc_sh.at[dst_v.at[j]], sem_s.at[b],
                         add=True)
        return 0
    lax.fori_loop(0, _NCHUNK, _chunk, 0)
    _wait_scat(_NCHUNK - 1, (_NCHUNK - 1) % 3)

    plsc.subcore_barrier()
    pltpu.sync_copy(acc_sh.at[pl.ds(sid * _SLC, _SLC)],
                    part.at[cid, pl.ds(sid * _SLC, _SLC)])


def _layer_call(table, fidx3, dst3, cidx3, norm_tab, zeros_hbm):
    return pl.kernel(
        _layer_body,
        out_type=jax.ShapeDtypeStruct((_NC, _ACC, _H), jnp.float32),
        mesh=_sc_mesh(),
        compiler_params=_SC_PARAMS,
        scratch_types=[
            pltpu.VMEM((_NCHUNK, _CH), jnp.int32),
            pltpu.VMEM((_NCHUNK, _CH), jnp.int32),
            pltpu.VMEM((_NCHUNK, _CH), jnp.int32),
            pltpu.VMEM((3, _CH, 16), jnp.float32),
            pltpu.VMEM((3, _CH, _H), jnp.float32),
            pltpu.SemaphoreType.DMA((3,)),
            pltpu.SemaphoreType.DMA((3,)),
            pltpu.SemaphoreType.DMA((3,)),
            pltpu.VMEM_SHARED((_ACC, _H), jnp.float32),
        ],
    )(table, fidx3, dst3, cidx3, norm_tab, zeros_hbm)


_BN = 1000


def _pre_body(x_ref, w_ref, root_ref, b_ref, t_ref, rp_ref):
    xb = x_ref[...]
    t_ref[...] = jnp.dot(xb, w_ref[...], preferred_element_type=jnp.float32)
    rp_ref[...] = (jnp.dot(xb, root_ref[...],
                           preferred_element_type=jnp.float32) + b_ref[...])


def _pre_call(x, wflat, root, b):
    f = x.shape[1]
    return pl.pallas_call(
        _pre_body,
        grid=(_N // _BN,),
        in_specs=[
            pl.BlockSpec((_BN, f), lambda i: (i, 0)),
            pl.BlockSpec((f, _R * _H), lambda i: (0, 0)),
            pl.BlockSpec((f, _H), lambda i: (0, 0)),
            pl.BlockSpec((1, _H), lambda i: (0, 0)),
        ],
        out_specs=[
            pl.BlockSpec((_BN, _R * _H), lambda i: (i, 0)),
            pl.BlockSpec((_BN, _H), lambda i: (i, 0)),
        ],
        out_shape=[
            jax.ShapeDtypeStruct((_N, _R * _H), jnp.float32),
            jax.ShapeDtypeStruct((_N, _H), jnp.float32),
        ],
    )(x, wflat, root, b)


def _mid_body(p0_ref, p1_ref, rp_ref, w_ref, root_ref, b_ref,
              h_ref, t_ref, rpn_ref):
    h = jnp.maximum(p0_ref[...] + p1_ref[...] + rp_ref[...], 0.0)
    h_ref[...] = h
    t_ref[...] = jnp.dot(h, w_ref[...], preferred_element_type=jnp.float32)
    rpn_ref[...] = (jnp.dot(h, root_ref[...],
                            preferred_element_type=jnp.float32) + b_ref[...])


def _mid_call(p0, p1, rp, wflat, root, b):
    return pl.pallas_call(
        _mid_body,
        grid=(_N // _BN,),
        in_specs=[
            pl.BlockSpec((_BN, _H), lambda i: (i, 0)),
            pl.BlockSpec((_BN, _H), lambda i: (i, 0)),
            pl.BlockSpec((_BN, _H), lambda i: (i, 0)),
            pl.BlockSpec((_H, _R * _H), lambda i: (0, 0)),
            pl.BlockSpec((_H, _H), lambda i: (0, 0)),
            pl.BlockSpec((1, _H), lambda i: (0, 0)),
        ],
        out_specs=[
            pl.BlockSpec((_BN, _H), lambda i: (i, 0)),
            pl.BlockSpec((_BN, _R * _H), lambda i: (i, 0)),
            pl.BlockSpec((_BN, _H), lambda i: (i, 0)),
        ],
        out_shape=[
            jax.ShapeDtypeStruct((_N, _H), jnp.float32),
            jax.ShapeDtypeStruct((_N, _R * _H), jnp.float32),
            jax.ShapeDtypeStruct((_N, _H), jnp.float32),
        ],
    )(p0, p1, rp, wflat, root, b)


def _post_body(p0_ref, p1_ref, rp_ref, h1_ref, h2_ref,
               lw1_ref, lw2_ref, lw3_ref, lb_ref, o_ref):
    h3 = jnp.maximum(p0_ref[...] + p1_ref[...] + rp_ref[...], 0.0)
    o_ref[...] = (
        jnp.dot(h1_ref[...], lw1_ref[...], preferred_element_type=jnp.float32)
        + jnp.dot(h2_ref[...], lw2_ref[...], preferred_element_type=jnp.float32)
        + jnp.dot(h3, lw3_ref[...], preferred_element_type=jnp.float32)
        + lb_ref[...])


def _post_call(p0, p1, rp, h1, h2, lw1, lw2, lw3, lb):
    return pl.pallas_call(
        _post_body,
        grid=(_N // _BN,),
        in_specs=[
            pl.BlockSpec((_BN, _H), lambda i: (i, 0)),
            pl.BlockSpec((_BN, _H), lambda i: (i, 0)),
            pl.BlockSpec((_BN, _H), lambda i: (i, 0)),
            pl.BlockSpec((_BN, _H), lambda i: (i, 0)),
            pl.BlockSpec((_BN, _H), lambda i: (i, 0)),
            pl.BlockSpec((_H, _C), lambda i: (0, 0)),
            pl.BlockSpec((_H, _C), lambda i: (0, 0)),
            pl.BlockSpec((_H, _C), lambda i: (0, 0)),
            pl.BlockSpec((1, _C), lambda i: (0, 0)),
        ],
        out_specs=pl.BlockSpec((_BN, _C), lambda i: (i, 0)),
        out_shape=jax.ShapeDtypeStruct((_N, _C), jnp.float32),
    )(p0, p1, rp, h1, h2, lw1, lw2, lw3, lb)


def kernel(x, edge_index, edge_type, w1, root1, b1, w2, root2, b2,
           w3, root3, b3, lin_w, lin_b):
    src = edge_index[0]
    dst = edge_index[1]

    src_p = jnp.concatenate([src, jnp.zeros((_PAD,), jnp.int32)])
    dst_p = jnp.concatenate([dst, jnp.full((_PAD,), _N, jnp.int32)])
    et_p = jnp.concatenate([edge_type, jnp.zeros((_PAD,), jnp.int32)])
    src3 = src_p.reshape(_NW, _NCHUNK, _CH)
    dst3 = dst_p.reshape(_NW, _NCHUNK, _CH)
    et3 = et_p.reshape(_NW, _NCHUNK, _CH)

    zeros_hbm = jnp.zeros((_ACC, _H), jnp.float32)
    zeros16 = jnp.zeros((_CNT, 16), jnp.float32)
    ones16 = jnp.ones((_CH, 16), jnp.float32)

    fidx3, cidx3, cnt_part = _p1_call(src3, dst3, et3, zeros16, ones16)
    norm_tab = _norm_call(cnt_part)

    w1f = w1.transpose(1, 0, 2).reshape(_F, _R * _H)
    w2f = w2.transpose(1, 0, 2).reshape(_H, _R * _H)
    w3f = w3.transpose(1, 0, 2).reshape(_H, _R * _H)

    t1, rp1 = _pre_call(x, w1f, root1, b1.reshape(1, _H))
    pt1 = _layer_call(t1.reshape(_N * _R, _H), fidx3, dst3, cidx3, norm_tab,
                      zeros_hbm)
    h1, t2, rp2 = _mid_call(pt1[0, :_N], pt1[1, :_N], rp1, w2f, root2,
                            b2.reshape(1, _H))
    pt2 = _layer_call(t2.reshape(_N * _R, _H), fidx3, dst3, cidx3, norm_tab,
                      zeros_hbm)
    h2, t3, rp3 = _mid_call(pt2[0, :_N], pt2[1, :_N], rp2, w3f, root3,
                            b3.reshape(1, _H))
    pt3 = _layer_call(t3.reshape(_N * _R, _H), fidx3, dst3, cidx3, norm_tab,
                      zeros_hbm)
    out = _post_call(pt3[0, :_N], pt3[1, :_N], rp3, h1, h2,
                     lin_w[:_H], lin_w[_H:2 * _H], lin_w[2 * _H:],
                     lin_b.reshape(1, _C))
    return out
